# initial kernel scaffold (unmeasured)
import jax
import jax.numpy as jnp
from jax import lax
from jax.experimental import pallas as pl
from jax.experimental.pallas import tpu as pltpu

N_DEV = 8
SQ = 1024
D = 1024
HD = 1024
DH = 128
KV0 = 1024
KV1 = 256
KV = KV0 + KV1
CH = SQ // N_DEV
SCALE = 0.08838834764831843
WIN = 128


def kernel(x, Wq, K_ext, V_ext, Wo):
    xs = x.reshape(SQ, D)
    K2 = K_ext.reshape(KV0, N_DEV * HD)
    V2 = V_ext.reshape(KV0, N_DEV * HD)

    def body(x_ref, wq_ref, k_hbm, v_hbm, wo_ref, out_ref,
             kbuf, vbuf, rbuf, loc, gsend, grecv, s1, r1, s2, r2):
        my_i = lax.axis_index("i")

        def g_rdma(t, src_hbm, dst_vmem, rows, j, sem_slot):
            return pltpu.make_async_remote_copy(
                src_ref=src_hbm.at[rows, pl.ds(j * HD, HD)],
                dst_ref=dst_vmem,
                send_sem=gsend.at[t, j],
                recv_sem=grecv.at[sem_slot],
                device_id=(j,),
                device_id_type=pl.DeviceIdType.MESH,
            )

        def dev0_rdmas():
            out = []
            for j in range(1, N_DEV):
                out.append(g_rdma(0, k_hbm, kbuf.at[0:KV0, :], slice(0, KV0), j, 0))
                out.append(g_rdma(1, v_hbm, vbuf.at[0:KV0, :], slice(0, KV0), j, 2))
            return out

        def dev1_rdmas():
            out = []
            for j in range(N_DEV):
                if j == 1:
                    continue
                out.append(g_rdma(0, k_hbm, kbuf.at[KV0:KV, :], slice(0, KV1), j, 1))
                out.append(g_rdma(1, v_hbm, vbuf.at[KV0:KV, :], slice(0, KV1), j, 3))
            return out

        @pl.when(my_i == 0)
        def _():
            pltpu.make_async_copy(
                k_hbm.at[:, 0:HD], kbuf.at[0:KV0, :], loc.at[0]).start()
            pltpu.make_async_copy(
                v_hbm.at[:, 0:HD], vbuf.at[0:KV0, :], loc.at[1]).start()
            for r in dev0_rdmas():
                r.start()

        @pl.when(my_i == 1)
        def _():
            pltpu.make_async_copy(
                k_hbm.at[0:KV1, pl.ds(HD, HD)], kbuf.at[KV0:KV, :], loc.at[0]).start()
            pltpu.make_async_copy(
                v_hbm.at[0:KV1, pl.ds(HD, HD)], vbuf.at[KV0:KV, :], loc.at[1]).start()
            for r in dev1_rdmas():
                r.start()

        q = jnp.dot(x_ref[...], wq_ref[...], preferred_element_type=jnp.float32)

        def wait_recv(dst, sem):
            pltpu.make_async_remote_copy(
                src_ref=dst, dst_ref=dst,
                send_sem=gsend.at[0, 0], recv_sem=sem,
                device_id=(0,), device_id_type=pl.DeviceIdType.MESH,
            ).wait_recv()

        @pl.when(jnp.logical_or(my_i == 0, my_i == 1))
        def _():
            pltpu.make_async_copy(k_hbm, kbuf, loc.at[0]).wait()
            pltpu.make_async_copy(k_hbm, kbuf, loc.at[1]).wait()

        @pl.when(my_i != 0)
        def _():
            wait_recv(kbuf.at[0:KV0, :], grecv.at[0])
            wait_recv(vbuf.at[0:KV0, :], grecv.at[2])

        @pl.when(my_i != 1)
        def _():
            wait_recv(kbuf.at[KV0:KV, :], grecv.at[1])
            wait_recv(vbuf.at[KV0:KV, :], grecv.at[3])

        kv = kbuf[...]
        vv = vbuf[...]
        qi = lax.broadcasted_iota(jnp.int32, (SQ, KV), 0)
        ki = lax.broadcasted_iota(jnp.int32, (SQ, KV), 1)
        mask = jnp.abs(qi - ki) <= WIN
        ctx_cols = []
        for h in range(HD // DH):
            qh = q[:, h * DH:(h + 1) * DH]
            kh = kv[:, h * DH:(h + 1) * DH]
            vh = vv[:, h * DH:(h + 1) * DH]
            s = lax.dot_general(
                qh, kh, (((1,), (1,)), ((), ())),
                preferred_element_type=jnp.float32) * SCALE
            s = jnp.where(mask, s, -1e9)
            m = jnp.max(s, axis=1, keepdims=True)
            w = jnp.exp(s - m)
            w = w / jnp.sum(w, axis=1, keepdims=True)
            ctx_cols.append(jnp.dot(w, vh, preferred_element_type=jnp.float32))
        ctx = jnp.concatenate(ctx_cols, axis=1)
        partial = jnp.dot(ctx, wo_ref[...], preferred_element_type=jnp.float32)
        out_ref[...] = partial

        rs = []
        for j in range(N_DEV):
            rdma = pltpu.make_async_remote_copy(
                src_ref=out_ref.at[pl.ds(j * CH, CH), :],
                dst_ref=rbuf.at[my_i],
                send_sem=s1.at[j],
                recv_sem=r1.at[my_i],
                device_id=(j,),
                device_id_type=pl.DeviceIdType.MESH,
            )
            rs.append((j, rdma))

            @pl.when(my_i != j)
            def _():
                rdma.start()

        own = out_ref[pl.ds(my_i * CH, CH), :]
        acc = jnp.zeros((CH, D), jnp.float32)
        for j in range(N_DEV):
            @pl.when(my_i != j)
            def _():
                pltpu.make_async_remote_copy(
                    src_ref=rbuf.at[j], dst_ref=rbuf.at[j],
                    send_sem=s1.at[j], recv_sem=r1.at[j],
                    device_id=(0,), device_id_type=pl.DeviceIdType.MESH,
                ).wait_recv()
            acc = acc + jnp.where(my_i == j, own, rbuf[j, :, :])
        out_ref[pl.ds(my_i * CH, CH), :] = acc

        ag = []
        for j in range(N_DEV):
            rdma = pltpu.make_async_remote_copy(
                src_ref=out_ref.at[pl.ds(my_i * CH, CH), :],
                dst_ref=out_ref.at[pl.ds(my_i * CH, CH), :],
                send_sem=s2.at[j],
                recv_sem=r2.at[my_i],
                device_id=(j,),
                device_id_type=pl.DeviceIdType.MESH,
            )
            ag.append((j, rdma))

            @pl.when(my_i != j)
            def _():
                rdma.start()

        for j in range(N_DEV):
            @pl.when(my_i != j)
            def _():
                pltpu.make_async_remote_copy(
                    src_ref=out_ref.at[pl.ds(j * CH, CH), :],
                    dst_ref=out_ref.at[pl.ds(j * CH, CH), :],
                    send_sem=s2.at[j], recv_sem=r2.at[j],
                    device_id=(0,), device_id_type=pl.DeviceIdType.MESH,
                ).wait_recv()

        for j, rdma in rs + ag:
            @pl.when(my_i != j)
            def _():
                rdma.wait_send()

        @pl.when(my_i == 0)
        def _():
            for r in dev0_rdmas():
                r.wait_send()

        @pl.when(my_i == 1)
        def _():
            for r in dev1_rdmas():
                r.wait_send()

    out = pl.pallas_call(
        body,
        out_shape=jax.ShapeDtypeStruct((SQ, D), jnp.float32),
        in_specs=[
            pl.BlockSpec(memory_space=pltpu.VMEM),
            pl.BlockSpec(memory_space=pltpu.VMEM),
            pl.BlockSpec(memory_space=pltpu.ANY),
            pl.BlockSpec(memory_space=pltpu.ANY),
            pl.BlockSpec(memory_space=pltpu.VMEM),
        ],
        out_specs=pl.BlockSpec(memory_space=pltpu.VMEM),
        scratch_shapes=[
            pltpu.VMEM((KV, HD), jnp.float32),
            pltpu.VMEM((KV, HD), jnp.float32),
            pltpu.VMEM((N_DEV, CH, D), jnp.float32),
            pltpu.SemaphoreType.DMA((2,)),
            pltpu.SemaphoreType.DMA((2, N_DEV)),
            pltpu.SemaphoreType.DMA((4,)),
            pltpu.SemaphoreType.DMA((N_DEV,)),
            pltpu.SemaphoreType.DMA((N_DEV,)),
            pltpu.SemaphoreType.DMA((N_DEV,)),
            pltpu.SemaphoreType.DMA((N_DEV,)),
        ],
        compiler_params=pltpu.CompilerParams(collective_id=0),
    )(xs, Wq, K2, V2, Wo)

    return out.reshape(1, SQ, D)


# baseline (device time: 515313 ns/iter reference)
import jax
import jax.numpy as jnp
from jax import lax
from jax.experimental import pallas as pl
from jax.experimental.pallas import tpu as pltpu

N_DEV = 8
SQ = 1024
D = 1024
HD = 1024
DH = 128
KV0 = 1024
KV1 = 256
KV = KV0 + KV1
CH = SQ // N_DEV
SCALE = 0.08838834764831843
WIN = 128


def kernel(x, Wq, K_ext, V_ext, Wo):
    xs = x.reshape(SQ, D)
    K2 = K_ext.reshape(KV0, N_DEV * HD)
    V2 = V_ext.reshape(KV0, N_DEV * HD)

    def body(x_ref, wq_ref, k_hbm, v_hbm, wo_ref, out_ref,
             kbuf, vbuf, rbuf, loc, gsend, grecv, s1, r1, s2, r2):
        my_i = lax.axis_index("i")

        def g_rdma(t, src_hbm, dst_vmem, rows, j, sem_slot):
            return pltpu.make_async_remote_copy(
                src_ref=src_hbm.at[rows, pl.ds(j * HD, HD)],
                dst_ref=dst_vmem,
                send_sem=gsend.at[t, j],
                recv_sem=grecv.at[sem_slot],
                device_id=(j,),
                device_id_type=pl.DeviceIdType.MESH,
            )

        def dev0_rdmas():
            out = []
            for j in range(1, N_DEV):
                out.append(g_rdma(0, k_hbm, kbuf.at[0:KV0, :], slice(0, KV0), j, 0))
                out.append(g_rdma(1, v_hbm, vbuf.at[0:KV0, :], slice(0, KV0), j, 2))
            return out

        def dev1_rdmas():
            out = []
            for j in range(N_DEV):
                if j == 1:
                    continue
                out.append(g_rdma(0, k_hbm, kbuf.at[KV0:KV, :], slice(0, KV1), j, 1))
                out.append(g_rdma(1, v_hbm, vbuf.at[KV0:KV, :], slice(0, KV1), j, 3))
            return out

        @pl.when(my_i == 0)
        def _():
            pltpu.make_async_copy(
                k_hbm.at[:, 0:HD], kbuf.at[0:KV0, :], loc.at[0]).start()
            pltpu.make_async_copy(
                v_hbm.at[:, 0:HD], vbuf.at[0:KV0, :], loc.at[1]).start()
            for r in dev0_rdmas():
                r.start()

        @pl.when(my_i == 1)
        def _():
            pltpu.make_async_copy(
                k_hbm.at[0:KV1, pl.ds(HD, HD)], kbuf.at[KV0:KV, :], loc.at[0]).start()
            pltpu.make_async_copy(
                v_hbm.at[0:KV1, pl.ds(HD, HD)], vbuf.at[KV0:KV, :], loc.at[1]).start()
            for r in dev1_rdmas():
                r.start()

        q = jnp.dot(x_ref[...], wq_ref[...], preferred_element_type=jnp.float32)

        def wait_recv(dst, sem):
            pltpu.make_async_remote_copy(
                src_ref=dst, dst_ref=dst,
                send_sem=gsend.at[0, 0], recv_sem=sem,
                device_id=(0,), device_id_type=pl.DeviceIdType.MESH,
            ).wait_recv()

        @pl.when(my_i == 0)
        def _():
            pltpu.make_async_copy(
                k_hbm.at[:, 0:HD], kbuf.at[0:KV0, :], loc.at[0]).wait()
            pltpu.make_async_copy(
                v_hbm.at[:, 0:HD], vbuf.at[0:KV0, :], loc.at[1]).wait()

        @pl.when(my_i == 1)
        def _():
            pltpu.make_async_copy(
                k_hbm.at[0:KV1, pl.ds(HD, HD)], kbuf.at[KV0:KV, :], loc.at[0]).wait()
            pltpu.make_async_copy(
                v_hbm.at[0:KV1, pl.ds(HD, HD)], vbuf.at[KV0:KV, :], loc.at[1]).wait()

        @pl.when(my_i != 0)
        def _():
            wait_recv(kbuf.at[0:KV0, :], grecv.at[0])
            wait_recv(vbuf.at[0:KV0, :], grecv.at[2])

        @pl.when(my_i != 1)
        def _():
            wait_recv(kbuf.at[KV0:KV, :], grecv.at[1])
            wait_recv(vbuf.at[KV0:KV, :], grecv.at[3])

        kv = kbuf[...]
        vv = vbuf[...]
        qi = lax.broadcasted_iota(jnp.int32, (SQ, KV), 0)
        ki = lax.broadcasted_iota(jnp.int32, (SQ, KV), 1)
        mask = jnp.abs(qi - ki) <= WIN
        ctx_cols = []
        for h in range(HD // DH):
            qh = q[:, h * DH:(h + 1) * DH]
            kh = kv[:, h * DH:(h + 1) * DH]
            vh = vv[:, h * DH:(h + 1) * DH]
            s = lax.dot_general(
                qh, kh, (((1,), (1,)), ((), ())),
                preferred_element_type=jnp.float32) * SCALE
            s = jnp.where(mask, s, -1e9)
            m = jnp.max(s, axis=1, keepdims=True)
            w = jnp.exp(s - m)
            w = w / jnp.sum(w, axis=1, keepdims=True)
            ctx_cols.append(jnp.dot(w, vh, preferred_element_type=jnp.float32))
        ctx = jnp.concatenate(ctx_cols, axis=1)
        partial = jnp.dot(ctx, wo_ref[...], preferred_element_type=jnp.float32)
        out_ref[...] = partial

        rs = []
        for j in range(N_DEV):
            rdma = pltpu.make_async_remote_copy(
                src_ref=out_ref.at[pl.ds(j * CH, CH), :],
                dst_ref=rbuf.at[my_i],
                send_sem=s1.at[j],
                recv_sem=r1.at[my_i],
                device_id=(j,),
                device_id_type=pl.DeviceIdType.MESH,
            )
            rs.append((j, rdma))

            @pl.when(my_i != j)
            def _():
                rdma.start()

        own = out_ref[pl.ds(my_i * CH, CH), :]
        acc = jnp.zeros((CH, D), jnp.float32)
        for j in range(N_DEV):
            @pl.when(my_i != j)
            def _():
                pltpu.make_async_remote_copy(
                    src_ref=rbuf.at[j], dst_ref=rbuf.at[j],
                    send_sem=s1.at[j], recv_sem=r1.at[j],
                    device_id=(0,), device_id_type=pl.DeviceIdType.MESH,
                ).wait_recv()
            acc = acc + jnp.where(my_i == j, own, rbuf[j, :, :])
        out_ref[pl.ds(my_i * CH, CH), :] = acc

        ag = []
        for j in range(N_DEV):
            rdma = pltpu.make_async_remote_copy(
                src_ref=out_ref.at[pl.ds(my_i * CH, CH), :],
                dst_ref=out_ref.at[pl.ds(my_i * CH, CH), :],
                send_sem=s2.at[j],
                recv_sem=r2.at[my_i],
                device_id=(j,),
                device_id_type=pl.DeviceIdType.MESH,
            )
            ag.append((j, rdma))

            @pl.when(my_i != j)
            def _():
                rdma.start()

        for j in range(N_DEV):
            @pl.when(my_i != j)
            def _():
                pltpu.make_async_remote_copy(
                    src_ref=out_ref.at[pl.ds(j * CH, CH), :],
                    dst_ref=out_ref.at[pl.ds(j * CH, CH), :],
                    send_sem=s2.at[j], recv_sem=r2.at[j],
                    device_id=(0,), device_id_type=pl.DeviceIdType.MESH,
                ).wait_recv()

        for j, rdma in rs + ag:
            @pl.when(my_i != j)
            def _():
                rdma.wait_send()

        @pl.when(my_i == 0)
        def _():
            for r in dev0_rdmas():
                r.wait_send()

        @pl.when(my_i == 1)
        def _():
            for r in dev1_rdmas():
                r.wait_send()

    out = pl.pallas_call(
        body,
        out_shape=jax.ShapeDtypeStruct((SQ, D), jnp.float32),
        in_specs=[
            pl.BlockSpec(memory_space=pltpu.VMEM),
            pl.BlockSpec(memory_space=pltpu.VMEM),
            pl.BlockSpec(memory_space=pl.ANY),
            pl.BlockSpec(memory_space=pl.ANY),
            pl.BlockSpec(memory_space=pltpu.VMEM),
        ],
        out_specs=pl.BlockSpec(memory_space=pltpu.VMEM),
        scratch_shapes=[
            pltpu.VMEM((KV, HD), jnp.float32),
            pltpu.VMEM((KV, HD), jnp.float32),
            pltpu.VMEM((N_DEV, CH, D), jnp.float32),
            pltpu.SemaphoreType.DMA((2,)),
            pltpu.SemaphoreType.DMA((2, N_DEV)),
            pltpu.SemaphoreType.DMA((4,)),
            pltpu.SemaphoreType.DMA((N_DEV,)),
            pltpu.SemaphoreType.DMA((N_DEV,)),
            pltpu.SemaphoreType.DMA((N_DEV,)),
            pltpu.SemaphoreType.DMA((N_DEV,)),
        ],
    )(xs, Wq, K2, V2, Wo)

    return out.reshape(1, SQ, D)


# device time: 371078 ns/iter; 1.3887x vs baseline; 1.3887x over previous
import jax
import jax.numpy as jnp
from jax import lax
from jax.experimental import pallas as pl
from jax.experimental.pallas import tpu as pltpu

N_DEV = 8
SQ = 1024
D = 1024
HD = 1024
DH = 128
KV0 = 1024
KV1 = 256
KV = KV0 + KV1
CH = SQ // N_DEV
QB = 128
KW = 384
SCALE = 0.08838834764831843
WIN = 128
STR = 128
BF = jnp.bfloat16


def kernel(x, Wq, K_ext, V_ext, Wo):
    xs = x.reshape(SQ, D)
    K2 = K_ext.reshape(KV0, N_DEV * HD)
    V2 = V_ext.reshape(KV0, N_DEV * HD)

    def body(x_ref, wq_ref, k_hbm, v_hbm, wo_ref, out_ref,
             kbuf, vbuf, stag, cbuf, sbuf, abuf, rbuf, gbuf,
             loc, gsend, grecv, s1, r1, s2, r2):
        my_i = lax.axis_index("i")

        def send_wave(src_hbm, dst_lo, dst_hi, nrows, grecv_slot, tsem):
            for c in range(nrows // STR):
                cp = pltpu.make_async_copy(
                    src_hbm.at[pl.ds(c * STR, STR), :],
                    stag.at[0:STR, :], loc.at[0])
                cp.start()
                cp.wait()
                cbuf[pl.ds(c * STR, STR), :] = stag[0:STR, :].astype(BF)
            rdmas = []
            for j in range(N_DEV):
                rdma = pltpu.make_async_remote_copy(
                    src_ref=cbuf.at[0:nrows, pl.ds(j * HD, HD)],
                    dst_ref=(kbuf if tsem == 0 else vbuf).at[dst_lo:dst_hi, :],
                    send_sem=gsend.at[tsem, j],
                    recv_sem=grecv.at[grecv_slot],
                    device_id=(j,),
                    device_id_type=pl.DeviceIdType.MESH,
                )
                @pl.when(my_i != j)
                def _():
                    rdma.start()
                rdmas.append((j, rdma))
            own = cbuf[0:nrows, pl.ds(my_i * HD, HD)]
            tgt = kbuf if tsem == 0 else vbuf
            tgt[dst_lo:dst_hi, :] = own
            for j, rdma in rdmas:
                @pl.when(my_i != j)
                def _():
                    rdma.wait_send()

        @pl.when(my_i == 0)
        def _():
            send_wave(k_hbm, 0, KV0, KV0, 0, 0)
            send_wave(v_hbm, 0, KV0, KV0, 2, 1)

        @pl.when(my_i == 1)
        def _():
            send_wave(k_hbm, KV0, KV, KV1, 1, 0)
            send_wave(v_hbm, KV0, KV, KV1, 3, 1)

        q = jnp.dot(x_ref[...].astype(BF), wq_ref[...].astype(BF),
                    preferred_element_type=jnp.float32).astype(BF)

        def wait_recv(dst, sem):
            pltpu.make_async_remote_copy(
                src_ref=dst, dst_ref=dst,
                send_sem=gsend.at[0, 0], recv_sem=sem,
                device_id=(0,), device_id_type=pl.DeviceIdType.MESH,
            ).wait_recv()

        @pl.when(my_i != 0)
        def _():
            wait_recv(kbuf.at[0:KV0, :], grecv.at[0])
            wait_recv(vbuf.at[0:KV0, :], grecv.at[2])

        @pl.when(my_i != 1)
        def _():
            wait_recv(kbuf.at[KV0:KV, :], grecv.at[1])
            wait_recv(vbuf.at[KV0:KV, :], grecv.at[3])

        strips = []
        for t in range(SQ // QB):
            start = max(0, (t - 1) * QB)
            qi = lax.broadcasted_iota(jnp.int32, (QB, KW), 0) + t * QB
            ki = lax.broadcasted_iota(jnp.int32, (QB, KW), 1) + start
            mask = jnp.abs(qi - ki) <= WIN
            blocks = []
            for h in range(HD // DH):
                qh = q[t * QB:(t + 1) * QB, h * DH:(h + 1) * DH]
                kh = kbuf[start:start + KW, h * DH:(h + 1) * DH]
                vh = vbuf[start:start + KW, h * DH:(h + 1) * DH]
                s = lax.dot_general(
                    qh, kh, (((1,), (1,)), ((), ())),
                    preferred_element_type=jnp.float32) * SCALE
                s = jnp.where(mask, s, -1e9)
                m = jnp.max(s, axis=1, keepdims=True)
                w = jnp.exp(s - m)
                w = w / jnp.sum(w, axis=1, keepdims=True)
                blocks.append(jnp.dot(w.astype(BF), vh,
                                      preferred_element_type=jnp.float32))
            strips.append(jnp.concatenate(blocks, axis=1))
        ctx = jnp.concatenate(strips, axis=0)
        partial = jnp.dot(ctx.astype(BF), wo_ref[...].astype(BF),
                          preferred_element_type=jnp.float32)
        out_ref[...] = partial
        sbuf[...] = partial.astype(BF)

        rs = []
        for j in range(N_DEV):
            rdma = pltpu.make_async_remote_copy(
                src_ref=sbuf.at[pl.ds(j * CH, CH), :],
                dst_ref=rbuf.at[my_i],
                send_sem=s1.at[j],
                recv_sem=r1.at[my_i],
                device_id=(j,),
                device_id_type=pl.DeviceIdType.MESH,
            )
            rs.append((j, rdma))

            @pl.when(my_i != j)
            def _():
                rdma.start()

        own = out_ref[pl.ds(my_i * CH, CH), :]
        acc = jnp.zeros((CH, D), jnp.float32)
        for j in range(N_DEV):
            @pl.when(my_i != j)
            def _():
                pltpu.make_async_remote_copy(
                    src_ref=rbuf.at[j], dst_ref=rbuf.at[j],
                    send_sem=s1.at[j], recv_sem=r1.at[j],
                    device_id=(0,), device_id_type=pl.DeviceIdType.MESH,
                ).wait_recv()
            acc = acc + jnp.where(my_i == j, own,
                                  rbuf[j, :, :].astype(jnp.float32))
        out_ref[pl.ds(my_i * CH, CH), :] = acc
        abuf[...] = acc.astype(BF)

        ag = []
        for j in range(N_DEV):
            rdma = pltpu.make_async_remote_copy(
                src_ref=abuf,
                dst_ref=gbuf.at[my_i],
                send_sem=s2.at[j],
                recv_sem=r2.at[my_i],
                device_id=(j,),
                device_id_type=pl.DeviceIdType.MESH,
            )
            ag.append((j, rdma))

            @pl.when(my_i != j)
            def _():
                rdma.start()

        for j in range(N_DEV):
            @pl.when(my_i != j)
            def _():
                pltpu.make_async_remote_copy(
                    src_ref=gbuf.at[j], dst_ref=gbuf.at[j],
                    send_sem=s2.at[j], recv_sem=r2.at[j],
                    device_id=(0,), device_id_type=pl.DeviceIdType.MESH,
                ).wait_recv()
                out_ref[pl.ds(j * CH, CH), :] = gbuf[j, :, :].astype(jnp.float32)

        for j, rdma in rs + ag:
            @pl.when(my_i != j)
            def _():
                rdma.wait_send()

    out = pl.pallas_call(
        body,
        out_shape=jax.ShapeDtypeStruct((SQ, D), jnp.float32),
        in_specs=[
            pl.BlockSpec(memory_space=pltpu.VMEM),
            pl.BlockSpec(memory_space=pltpu.VMEM),
            pl.BlockSpec(memory_space=pl.ANY),
            pl.BlockSpec(memory_space=pl.ANY),
            pl.BlockSpec(memory_space=pltpu.VMEM),
        ],
        out_specs=pl.BlockSpec(memory_space=pltpu.VMEM),
        scratch_shapes=[
            pltpu.VMEM((KV, HD), BF),
            pltpu.VMEM((KV, HD), BF),
            pltpu.VMEM((STR, N_DEV * HD), jnp.float32),
            pltpu.VMEM((KV0, N_DEV * HD), BF),
            pltpu.VMEM((SQ, D), BF),
            pltpu.VMEM((CH, D), BF),
            pltpu.VMEM((N_DEV, CH, D), BF),
            pltpu.VMEM((N_DEV, CH, D), BF),
            pltpu.SemaphoreType.DMA((1,)),
            pltpu.SemaphoreType.DMA((2, N_DEV)),
            pltpu.SemaphoreType.DMA((4,)),
            pltpu.SemaphoreType.DMA((N_DEV,)),
            pltpu.SemaphoreType.DMA((N_DEV,)),
            pltpu.SemaphoreType.DMA((N_DEV,)),
            pltpu.SemaphoreType.DMA((N_DEV,)),
        ],
        compiler_params=pltpu.CompilerParams(
            vmem_limit_bytes=60 * 1024 * 1024,
        ),
    )(xs, Wq, K2, V2, Wo)

    return out.reshape(1, SQ, D)


# device time: 333079 ns/iter; 1.5471x vs baseline; 1.1141x over previous
import jax
import jax.numpy as jnp
from jax import lax
from jax.experimental import pallas as pl
from jax.experimental.pallas import tpu as pltpu

N_DEV = 8
SQ = 1024
D = 1024
HD = 1024
DH = 128
KV0 = 1024
KV1 = 256
KV = KV0 + KV1
CH = SQ // N_DEV
QB = 128
KW = 384
SCALE = 0.08838834764831843
WIN = 128
STR = 256
BF = jnp.bfloat16


def kernel(x, Wq, K_ext, V_ext, Wo):
    xs = x.reshape(SQ, D)
    K2 = K_ext.reshape(KV0, N_DEV * HD)
    V2 = V_ext.reshape(KV0, N_DEV * HD)

    def body(x_ref, wq_ref, k_hbm, v_hbm, wo_ref, out_ref,
             kbuf, vbuf, stag, cbuf, sbuf, abuf, rbuf, gbuf,
             loc, gsend, grecv, s1, r1, s2, r2):
        my_i = lax.axis_index("i")

        def load_convert(hbm, c, par):
            cp = pltpu.make_async_copy(
                hbm.at[pl.ds(c * STR, STR), :], stag, loc.at[0])
            cp.start()
            cp.wait()
            cbuf[par, :, :] = stag[...].astype(BF)

        @pl.when(my_i == 0)
        def _():
            prev = [[None] * N_DEV, [None] * N_DEV]
            for cc in range(8):
                t, c = cc // 4, cc % 4
                par = cc % 2
                hbm = k_hbm if t == 0 else v_hbm
                buf = kbuf if t == 0 else vbuf
                for j in range(1, N_DEV):
                    if prev[par][j] is not None:
                        prev[par][j].wait_send()
                load_convert(hbm, c, par)
                for j in range(1, N_DEV):
                    rdma = pltpu.make_async_remote_copy(
                        src_ref=cbuf.at[par, :, pl.ds(j * HD, HD)],
                        dst_ref=buf.at[c * STR:(c + 1) * STR, :],
                        send_sem=gsend.at[par, j],
                        recv_sem=grecv.at[cc],
                        device_id=(j,),
                        device_id_type=pl.DeviceIdType.MESH,
                    )
                    rdma.start()
                    prev[par][j] = rdma
                buf[c * STR:(c + 1) * STR, :] = cbuf[par, :, 0:HD]
            for par in range(2):
                for j in range(1, N_DEV):
                    prev[par][j].wait_send()

        @pl.when(my_i == 1)
        def _():
            for t in range(2):
                hbm = k_hbm if t == 0 else v_hbm
                buf = kbuf if t == 0 else vbuf
                load_convert(hbm, 0, t)
                rdmas = []
                for j in range(N_DEV):
                    if j == 1:
                        continue
                    rdma = pltpu.make_async_remote_copy(
                        src_ref=cbuf.at[t, :, pl.ds(j * HD, HD)],
                        dst_ref=buf.at[KV0:KV, :],
                        send_sem=gsend.at[t, j],
                        recv_sem=grecv.at[8 + t],
                        device_id=(j,),
                        device_id_type=pl.DeviceIdType.MESH,
                    )
                    rdma.start()
                    rdmas.append(rdma)
                buf[KV0:KV, :] = cbuf[t, :, HD:2 * HD]
                for rdma in rdmas:
                    rdma.wait_send()

        q = jnp.dot(x_ref[...].astype(BF), wq_ref[...].astype(BF),
                    preferred_element_type=jnp.float32).astype(BF)

        def wait_recv(dst, sem):
            pltpu.make_async_remote_copy(
                src_ref=dst, dst_ref=dst,
                send_sem=gsend.at[0, 0], recv_sem=sem,
                device_id=(0,), device_id_type=pl.DeviceIdType.MESH,
            ).wait_recv()

        @pl.when(my_i != 0)
        def _():
            for cc in range(8):
                t, c = cc // 4, cc % 4
                buf = kbuf if t == 0 else vbuf
                wait_recv(buf.at[c * STR:(c + 1) * STR, :], grecv.at[cc])

        @pl.when(my_i != 1)
        def _():
            wait_recv(kbuf.at[KV0:KV, :], grecv.at[8])
            wait_recv(vbuf.at[KV0:KV, :], grecv.at[9])

        strips = []
        for t in range(SQ // QB):
            start = max(0, (t - 1) * QB)
            qi = lax.broadcasted_iota(jnp.int32, (QB, KW), 0) + t * QB
            ki = lax.broadcasted_iota(jnp.int32, (QB, KW), 1) + start
            mask = jnp.abs(qi - ki) <= WIN
            blocks = []
            for h in range(HD // DH):
                qh = q[t * QB:(t + 1) * QB, h * DH:(h + 1) * DH]
                kh = kbuf[start:start + KW, h * DH:(h + 1) * DH]
                vh = vbuf[start:start + KW, h * DH:(h + 1) * DH]
                s = lax.dot_general(
                    qh, kh, (((1,), (1,)), ((), ())),
                    preferred_element_type=jnp.float32) * SCALE
                s = jnp.where(mask, s, -1e9)
                m = jnp.max(s, axis=1, keepdims=True)
                w = jnp.exp(s - m)
                w = w / jnp.sum(w, axis=1, keepdims=True)
                blocks.append(jnp.dot(w.astype(BF), vh,
                                      preferred_element_type=jnp.float32))
            strips.append(jnp.concatenate(blocks, axis=1))
        ctx = jnp.concatenate(strips, axis=0)
        partial = jnp.dot(ctx.astype(BF), wo_ref[...].astype(BF),
                          preferred_element_type=jnp.float32)
        out_ref[...] = partial
        sbuf[...] = partial.astype(BF)

        rs = []
        for j in range(N_DEV):
            rdma = pltpu.make_async_remote_copy(
                src_ref=sbuf.at[pl.ds(j * CH, CH), :],
                dst_ref=rbuf.at[my_i],
                send_sem=s1.at[j],
                recv_sem=r1.at[my_i],
                device_id=(j,),
                device_id_type=pl.DeviceIdType.MESH,
            )
            rs.append((j, rdma))

            @pl.when(my_i != j)
            def _():
                rdma.start()

        own = out_ref[pl.ds(my_i * CH, CH), :]
        acc = jnp.zeros((CH, D), jnp.float32)
        for j in range(N_DEV):
            @pl.when(my_i != j)
            def _():
                pltpu.make_async_remote_copy(
                    src_ref=rbuf.at[j], dst_ref=rbuf.at[j],
                    send_sem=s1.at[j], recv_sem=r1.at[j],
                    device_id=(0,), device_id_type=pl.DeviceIdType.MESH,
                ).wait_recv()
            acc = acc + jnp.where(my_i == j, own,
                                  rbuf[j, :, :].astype(jnp.float32))
        out_ref[pl.ds(my_i * CH, CH), :] = acc
        abuf[...] = acc.astype(BF)

        ag = []
        for j in range(N_DEV):
            rdma = pltpu.make_async_remote_copy(
                src_ref=abuf,
                dst_ref=gbuf.at[my_i],
                send_sem=s2.at[j],
                recv_sem=r2.at[my_i],
                device_id=(j,),
                device_id_type=pl.DeviceIdType.MESH,
            )
            ag.append((j, rdma))

            @pl.when(my_i != j)
            def _():
                rdma.start()

        for j in range(N_DEV):
            @pl.when(my_i != j)
            def _():
                pltpu.make_async_remote_copy(
                    src_ref=gbuf.at[j], dst_ref=gbuf.at[j],
                    send_sem=s2.at[j], recv_sem=r2.at[j],
                    device_id=(0,), device_id_type=pl.DeviceIdType.MESH,
                ).wait_recv()
                out_ref[pl.ds(j * CH, CH), :] = gbuf[j, :, :].astype(jnp.float32)

        for j, rdma in rs + ag:
            @pl.when(my_i != j)
            def _():
                rdma.wait_send()

    out = pl.pallas_call(
        body,
        out_shape=jax.ShapeDtypeStruct((SQ, D), jnp.float32),
        in_specs=[
            pl.BlockSpec(memory_space=pltpu.VMEM),
            pl.BlockSpec(memory_space=pltpu.VMEM),
            pl.BlockSpec(memory_space=pl.ANY),
            pl.BlockSpec(memory_space=pl.ANY),
            pl.BlockSpec(memory_space=pltpu.VMEM),
        ],
        out_specs=pl.BlockSpec(memory_space=pltpu.VMEM),
        scratch_shapes=[
            pltpu.VMEM((KV, HD), BF),
            pltpu.VMEM((KV, HD), BF),
            pltpu.VMEM((STR, N_DEV * HD), jnp.float32),
            pltpu.VMEM((2, STR, N_DEV * HD), BF),
            pltpu.VMEM((SQ, D), BF),
            pltpu.VMEM((CH, D), BF),
            pltpu.VMEM((N_DEV, CH, D), BF),
            pltpu.VMEM((N_DEV, CH, D), BF),
            pltpu.SemaphoreType.DMA((1,)),
            pltpu.SemaphoreType.DMA((2, N_DEV)),
            pltpu.SemaphoreType.DMA((10,)),
            pltpu.SemaphoreType.DMA((N_DEV,)),
            pltpu.SemaphoreType.DMA((N_DEV,)),
            pltpu.SemaphoreType.DMA((N_DEV,)),
            pltpu.SemaphoreType.DMA((N_DEV,)),
        ],
        compiler_params=pltpu.CompilerParams(
            vmem_limit_bytes=60 * 1024 * 1024,
        ),
    )(xs, Wq, K2, V2, Wo)

    return out.reshape(1, SQ, D)


# device time: 266533 ns/iter; 1.9334x vs baseline; 1.2497x over previous
import jax
import jax.numpy as jnp
from jax import lax
from jax.experimental import pallas as pl
from jax.experimental.pallas import tpu as pltpu

N_DEV = 8
SQ = 1024
D = 1024
HD = 1024
DH = 128
KV0 = 1024
KV1 = 256
KV = KV0 + KV1
CH = SQ // N_DEV
QB = 128
KW = 384
SCALE = 0.08838834764831843
WIN = 128
STR = 256
BF = jnp.bfloat16


def kernel(x, Wq, K_ext, V_ext, Wo):
    xs = x.reshape(SQ, D)

    def body(x_ref, wq_ref, k_hbm, v_hbm, wo_ref, out_ref,
             kbuf, vbuf, stag, cbuf, sbuf, abuf, rbuf, gbuf,
             loc, gsend, grecv, s1, r1, s2, r2):
        my_i = lax.axis_index("i")

        def load_convert(hbm, c, par):
            cp = pltpu.make_async_copy(
                hbm.at[0, pl.ds(c * STR, STR), :, :], stag, loc.at[0])
            cp.start()
            cp.wait()
            cbuf[par, :, :, :] = stag[...].astype(BF)

        @pl.when(my_i == 0)
        def _():
            prev = [[None] * N_DEV, [None] * N_DEV]
            for cc in range(8):
                t, c = cc // 4, cc % 4
                par = cc % 2
                hbm = k_hbm if t == 0 else v_hbm
                buf = kbuf if t == 0 else vbuf
                for j in range(1, N_DEV):
                    if prev[par][j] is not None:
                        prev[par][j].wait_send()
                load_convert(hbm, c, par)
                for j in range(1, N_DEV):
                    rdma = pltpu.make_async_remote_copy(
                        src_ref=cbuf.at[par, :, pl.ds(j * 8, 8), :],
                        dst_ref=buf.at[c * STR:(c + 1) * STR, :, :],
                        send_sem=gsend.at[par, j],
                        recv_sem=grecv.at[cc],
                        device_id=(j,),
                        device_id_type=pl.DeviceIdType.MESH,
                    )
                    rdma.start()
                    prev[par][j] = rdma
                buf[c * STR:(c + 1) * STR, :, :] = cbuf[par, :, 0:8, :]
            for par in range(2):
                for j in range(1, N_DEV):
                    prev[par][j].wait_send()

        @pl.when(my_i == 1)
        def _():
            for t in range(2):
                hbm = k_hbm if t == 0 else v_hbm
                buf = kbuf if t == 0 else vbuf
                load_convert(hbm, 0, t)
                rdmas = []
                for j in range(N_DEV):
                    if j == 1:
                        continue
                    rdma = pltpu.make_async_remote_copy(
                        src_ref=cbuf.at[t, :, pl.ds(j * 8, 8), :],
                        dst_ref=buf.at[KV0:KV, :, :],
                        send_sem=gsend.at[t, j],
                        recv_sem=grecv.at[8 + t],
                        device_id=(j,),
                        device_id_type=pl.DeviceIdType.MESH,
                    )
                    rdma.start()
                    rdmas.append(rdma)
                buf[KV0:KV, :, :] = cbuf[t, :, 8:16, :]
                for rdma in rdmas:
                    rdma.wait_send()

        q = jnp.dot(x_ref[...].astype(BF), wq_ref[...].astype(BF),
                    preferred_element_type=jnp.float32).astype(BF)

        def wait_recv(dst, sem):
            pltpu.make_async_remote_copy(
                src_ref=dst, dst_ref=dst,
                send_sem=gsend.at[0, 0], recv_sem=sem,
                device_id=(0,), device_id_type=pl.DeviceIdType.MESH,
            ).wait_recv()

        @pl.when(my_i != 0)
        def _():
            for cc in range(8):
                t, c = cc // 4, cc % 4
                buf = kbuf if t == 0 else vbuf
                wait_recv(buf.at[c * STR:(c + 1) * STR, :, :], grecv.at[cc])

        @pl.when(my_i != 1)
        def _():
            wait_recv(kbuf.at[KV0:KV, :, :], grecv.at[8])
            wait_recv(vbuf.at[KV0:KV, :, :], grecv.at[9])

        strips = []
        for t in range(SQ // QB):
            start = max(0, (t - 1) * QB)
            qi = lax.broadcasted_iota(jnp.int32, (QB, KW), 0) + t * QB
            ki = lax.broadcasted_iota(jnp.int32, (QB, KW), 1) + start
            mask = jnp.abs(qi - ki) <= WIN
            blocks = []
            for h in range(HD // DH):
                qh = q[t * QB:(t + 1) * QB, h * DH:(h + 1) * DH]
                kh = kbuf[start:start + KW, h, :]
                vh = vbuf[start:start + KW, h, :]
                s = lax.dot_general(
                    qh, kh, (((1,), (1,)), ((), ())),
                    preferred_element_type=jnp.float32) * SCALE
                s = jnp.where(mask, s, -1e9)
                m = jnp.max(s, axis=1, keepdims=True)
                w = jnp.exp(s - m)
                w = w / jnp.sum(w, axis=1, keepdims=True)
                blocks.append(jnp.dot(w.astype(BF), vh,
                                      preferred_element_type=jnp.float32))
            strips.append(jnp.concatenate(blocks, axis=1))
        ctx = jnp.concatenate(strips, axis=0)
        partial = jnp.dot(ctx.astype(BF), wo_ref[...].astype(BF),
                          preferred_element_type=jnp.float32)
        out_ref[...] = partial
        sbuf[...] = partial.astype(BF)

        rs = []
        for j in range(N_DEV):
            rdma = pltpu.make_async_remote_copy(
                src_ref=sbuf.at[pl.ds(j * CH, CH), :],
                dst_ref=rbuf.at[my_i],
                send_sem=s1.at[j],
                recv_sem=r1.at[my_i],
                device_id=(j,),
                device_id_type=pl.DeviceIdType.MESH,
            )
            rs.append((j, rdma))

            @pl.when(my_i != j)
            def _():
                rdma.start()

        own = out_ref[pl.ds(my_i * CH, CH), :]
        acc = jnp.zeros((CH, D), jnp.float32)
        for j in range(N_DEV):
            @pl.when(my_i != j)
            def _():
                pltpu.make_async_remote_copy(
                    src_ref=rbuf.at[j], dst_ref=rbuf.at[j],
                    send_sem=s1.at[j], recv_sem=r1.at[j],
                    device_id=(0,), device_id_type=pl.DeviceIdType.MESH,
                ).wait_recv()
            acc = acc + jnp.where(my_i == j, own,
                                  rbuf[j, :, :].astype(jnp.float32))
        out_ref[pl.ds(my_i * CH, CH), :] = acc
        abuf[...] = acc.astype(BF)

        ag = []
        for j in range(N_DEV):
            rdma = pltpu.make_async_remote_copy(
                src_ref=abuf,
                dst_ref=gbuf.at[my_i],
                send_sem=s2.at[j],
                recv_sem=r2.at[my_i],
                device_id=(j,),
                device_id_type=pl.DeviceIdType.MESH,
            )
            ag.append((j, rdma))

            @pl.when(my_i != j)
            def _():
                rdma.start()

        for j in range(N_DEV):
            @pl.when(my_i != j)
            def _():
                pltpu.make_async_remote_copy(
                    src_ref=gbuf.at[j], dst_ref=gbuf.at[j],
                    send_sem=s2.at[j], recv_sem=r2.at[j],
                    device_id=(0,), device_id_type=pl.DeviceIdType.MESH,
                ).wait_recv()
                out_ref[pl.ds(j * CH, CH), :] = gbuf[j, :, :].astype(jnp.float32)

        for j, rdma in rs + ag:
            @pl.when(my_i != j)
            def _():
                rdma.wait_send()

    out = pl.pallas_call(
        body,
        out_shape=jax.ShapeDtypeStruct((SQ, D), jnp.float32),
        in_specs=[
            pl.BlockSpec(memory_space=pltpu.VMEM),
            pl.BlockSpec(memory_space=pltpu.VMEM),
            pl.BlockSpec(memory_space=pl.ANY),
            pl.BlockSpec(memory_space=pl.ANY),
            pl.BlockSpec(memory_space=pltpu.VMEM),
        ],
        out_specs=pl.BlockSpec(memory_space=pltpu.VMEM),
        scratch_shapes=[
            pltpu.VMEM((KV, 8, DH), BF),
            pltpu.VMEM((KV, 8, DH), BF),
            pltpu.VMEM((STR, 64, DH), jnp.float32),
            pltpu.VMEM((2, STR, 64, DH), BF),
            pltpu.VMEM((SQ, D), BF),
            pltpu.VMEM((CH, D), BF),
            pltpu.VMEM((N_DEV, CH, D), BF),
            pltpu.VMEM((N_DEV, CH, D), BF),
            pltpu.SemaphoreType.DMA((1,)),
            pltpu.SemaphoreType.DMA((2, N_DEV)),
            pltpu.SemaphoreType.DMA((10,)),
            pltpu.SemaphoreType.DMA((N_DEV,)),
            pltpu.SemaphoreType.DMA((N_DEV,)),
            pltpu.SemaphoreType.DMA((N_DEV,)),
            pltpu.SemaphoreType.DMA((N_DEV,)),
        ],
        compiler_params=pltpu.CompilerParams(
            vmem_limit_bytes=60 * 1024 * 1024,
        ),
    )(xs, Wq, K_ext, V_ext, Wo)

    return out.reshape(1, SQ, D)


# device time: 256048 ns/iter; 2.0126x vs baseline; 1.0409x over previous
import jax
import jax.numpy as jnp
from jax import lax
from jax.experimental import pallas as pl
from jax.experimental.pallas import tpu as pltpu

N_DEV = 8
SQ = 1024
D = 1024
HD = 1024
DH = 128
KV0 = 1024
KV1 = 256
KV = KV0 + KV1
CH = SQ // N_DEV
QB = 128
KW = 384
SCALE = 0.08838834764831843
WIN = 128
STR = 256
BF = jnp.bfloat16


def kernel(x, Wq, K_ext, V_ext, Wo):
    xs = x.reshape(SQ, D)

    def body(x_ref, wq_ref, k_hbm, v_hbm, wo_ref, out_ref,
             kbuf, vbuf, stag, cbuf, sbuf, abuf, rbuf, gbuf,
             loc, gsend, grecv, s1, r1, s2, r2):
        my_i = lax.axis_index("i")

        def load_convert(hbm, c, par):
            cp = pltpu.make_async_copy(
                hbm.at[0, pl.ds(c * STR, STR), :, :], stag, loc.at[0])
            cp.start()
            cp.wait()
            cbuf[par, :, :, :] = stag[...].astype(BF)

        @pl.when(my_i == 0)
        def _():
            prev = [[None] * N_DEV, [None] * N_DEV]
            for cc in range(8):
                t, c = cc % 2, cc // 2
                par = cc % 2
                hbm = k_hbm if t == 0 else v_hbm
                buf = kbuf if t == 0 else vbuf
                for j in range(1, N_DEV):
                    if prev[par][j] is not None:
                        prev[par][j].wait_send()
                load_convert(hbm, c, par)
                for j in range(1, N_DEV):
                    rdma = pltpu.make_async_remote_copy(
                        src_ref=cbuf.at[par, :, pl.ds(j * 8, 8), :],
                        dst_ref=buf.at[c * STR:(c + 1) * STR, :, :],
                        send_sem=gsend.at[par, j],
                        recv_sem=grecv.at[cc],
                        device_id=(j,),
                        device_id_type=pl.DeviceIdType.MESH,
                    )
                    rdma.start()
                    prev[par][j] = rdma
                buf[c * STR:(c + 1) * STR, :, :] = cbuf[par, :, 0:8, :]
            for par in range(2):
                for j in range(1, N_DEV):
                    prev[par][j].wait_send()

        @pl.when(my_i == 1)
        def _():
            for t in range(2):
                hbm = k_hbm if t == 0 else v_hbm
                buf = kbuf if t == 0 else vbuf
                load_convert(hbm, 0, t)
                rdmas = []
                for j in range(N_DEV):
                    if j == 1:
                        continue
                    rdma = pltpu.make_async_remote_copy(
                        src_ref=cbuf.at[t, :, pl.ds(j * 8, 8), :],
                        dst_ref=buf.at[KV0:KV, :, :],
                        send_sem=gsend.at[t, j],
                        recv_sem=grecv.at[8 + t],
                        device_id=(j,),
                        device_id_type=pl.DeviceIdType.MESH,
                    )
                    rdma.start()
                    rdmas.append(rdma)
                buf[KV0:KV, :, :] = cbuf[t, :, 8:16, :]
                for rdma in rdmas:
                    rdma.wait_send()

        q = jnp.dot(x_ref[...].astype(BF), wq_ref[...].astype(BF),
                    preferred_element_type=jnp.float32).astype(BF)

        def wait_recv(dst, sem):
            pltpu.make_async_remote_copy(
                src_ref=dst, dst_ref=dst,
                send_sem=gsend.at[0, 0], recv_sem=sem,
                device_id=(0,), device_id_type=pl.DeviceIdType.MESH,
            ).wait_recv()

        @pl.when(my_i != 1)
        def _():
            wait_recv(kbuf.at[KV0:KV, :, :], grecv.at[8])
            wait_recv(vbuf.at[KV0:KV, :, :], grecv.at[9])

        wo_bf = wo_ref[...].astype(BF)

        ENABLED = {0: [0], 1: [1, 2], 2: [3, 4], 3: [5, 6, 7]}
        rs = []
        for c in range(4):
            @pl.when(my_i != 0)
            def _():
                wait_recv(kbuf.at[c * STR:(c + 1) * STR, :, :], grecv.at[2 * c])
                wait_recv(vbuf.at[c * STR:(c + 1) * STR, :, :], grecv.at[2 * c + 1])
            for t in ENABLED[c]:
                start = max(0, (t - 1) * QB)
                kw = 2 * QB if t == 0 else KW
                qi = lax.broadcasted_iota(jnp.int32, (QB, kw), 0) + t * QB
                ki = lax.broadcasted_iota(jnp.int32, (QB, kw), 1) + start
                mask = jnp.abs(qi - ki) <= WIN
                blocks = []
                for h in range(HD // DH):
                    qh = q[t * QB:(t + 1) * QB, h * DH:(h + 1) * DH]
                    kh = kbuf[start:start + kw, h, :]
                    vh = vbuf[start:start + kw, h, :]
                    sc = lax.dot_general(
                        qh, kh, (((1,), (1,)), ((), ())),
                        preferred_element_type=jnp.float32) * SCALE
                    sc = jnp.where(mask, sc, -1e9)
                    m = jnp.max(sc, axis=1, keepdims=True)
                    w = jnp.exp(sc - m)
                    w = w / jnp.sum(w, axis=1, keepdims=True)
                    blocks.append(jnp.dot(w.astype(BF), vh,
                                          preferred_element_type=jnp.float32))
                strip = jnp.concatenate(blocks, axis=1)
                partial_t = jnp.dot(strip.astype(BF), wo_bf,
                                    preferred_element_type=jnp.float32)
                out_ref[t * QB:(t + 1) * QB, :] = partial_t
                sbuf[t * QB:(t + 1) * QB, :] = partial_t.astype(BF)
                rdma = pltpu.make_async_remote_copy(
                    src_ref=sbuf.at[t * QB:(t + 1) * QB, :],
                    dst_ref=rbuf.at[my_i],
                    send_sem=s1.at[t],
                    recv_sem=r1.at[my_i],
                    device_id=(t,),
                    device_id_type=pl.DeviceIdType.MESH,
                )
                rs.append((t, rdma))

                @pl.when(my_i != t)
                def _():
                    rdma.start()

        own = out_ref[pl.ds(my_i * CH, CH), :]
        acc = jnp.zeros((CH, D), jnp.float32)
        for j in range(N_DEV):
            @pl.when(my_i != j)
            def _():
                pltpu.make_async_remote_copy(
                    src_ref=rbuf.at[j], dst_ref=rbuf.at[j],
                    send_sem=s1.at[j], recv_sem=r1.at[j],
                    device_id=(0,), device_id_type=pl.DeviceIdType.MESH,
                ).wait_recv()
            acc = acc + jnp.where(my_i == j, own,
                                  rbuf[j, :, :].astype(jnp.float32))
        out_ref[pl.ds(my_i * CH, CH), :] = acc
        abuf[...] = acc.astype(BF)

        ag = []
        for j in range(N_DEV):
            rdma = pltpu.make_async_remote_copy(
                src_ref=abuf,
                dst_ref=gbuf.at[my_i],
                send_sem=s2.at[j],
                recv_sem=r2.at[my_i],
                device_id=(j,),
                device_id_type=pl.DeviceIdType.MESH,
            )
            ag.append((j, rdma))

            @pl.when(my_i != j)
            def _():
                rdma.start()

        for j in range(N_DEV):
            @pl.when(my_i != j)
            def _():
                pltpu.make_async_remote_copy(
                    src_ref=gbuf.at[j], dst_ref=gbuf.at[j],
                    send_sem=s2.at[j], recv_sem=r2.at[j],
                    device_id=(0,), device_id_type=pl.DeviceIdType.MESH,
                ).wait_recv()
                out_ref[pl.ds(j * CH, CH), :] = gbuf[j, :, :].astype(jnp.float32)

        for j, rdma in rs + ag:
            @pl.when(my_i != j)
            def _():
                rdma.wait_send()

    out = pl.pallas_call(
        body,
        out_shape=jax.ShapeDtypeStruct((SQ, D), jnp.float32),
        in_specs=[
            pl.BlockSpec(memory_space=pltpu.VMEM),
            pl.BlockSpec(memory_space=pltpu.VMEM),
            pl.BlockSpec(memory_space=pl.ANY),
            pl.BlockSpec(memory_space=pl.ANY),
            pl.BlockSpec(memory_space=pltpu.VMEM),
        ],
        out_specs=pl.BlockSpec(memory_space=pltpu.VMEM),
        scratch_shapes=[
            pltpu.VMEM((KV, 8, DH), BF),
            pltpu.VMEM((KV, 8, DH), BF),
            pltpu.VMEM((STR, 64, DH), jnp.float32),
            pltpu.VMEM((2, STR, 64, DH), BF),
            pltpu.VMEM((SQ, D), BF),
            pltpu.VMEM((CH, D), BF),
            pltpu.VMEM((N_DEV, CH, D), BF),
            pltpu.VMEM((N_DEV, CH, D), BF),
            pltpu.SemaphoreType.DMA((1,)),
            pltpu.SemaphoreType.DMA((2, N_DEV)),
            pltpu.SemaphoreType.DMA((10,)),
            pltpu.SemaphoreType.DMA((N_DEV,)),
            pltpu.SemaphoreType.DMA((N_DEV,)),
            pltpu.SemaphoreType.DMA((N_DEV,)),
            pltpu.SemaphoreType.DMA((N_DEV,)),
        ],
        compiler_params=pltpu.CompilerParams(
            vmem_limit_bytes=60 * 1024 * 1024,
        ),
    )(xs, Wq, K_ext, V_ext, Wo)

    return out.reshape(1, SQ, D)
